# Initial kernel scaffold; baseline (speedup 1.0000x reference)
import jax
import jax.numpy as jnp
from jax.experimental import pallas as pl


def kernel(x, batch_idx, dim_size, W1, b1, W2, b2):
    raise NotImplementedError("write your pallas kernel here")


# speedup vs baseline: 5.5104x; 5.5104x over previous
import functools

import jax
import jax.numpy as jnp
from jax import lax
from jax.experimental import pallas as pl
from jax.experimental.pallas import tpu as pltpu
from jax.experimental.pallas import tpu_sc as plsc

N = 100000
D = 128
H = 128
G = 512
B = 2048
NP = 100352
NB = NP // B
NSC = 32
CHUNK = NP // NSC
LANES = 16


def _tc_body(x_ref, idx_ref, w1_ref, b1_ref, w2_ref, b2_ref,
             ex_ref, out_ref, denom_ref, dacc):
    i = pl.program_id(0)
    x = x_ref[...]
    h = jnp.tanh(
        lax.dot_general(x, w1_ref[...], (((1,), (0,)), ((), ())),
                        preferred_element_type=jnp.float32) + b1_ref[...])
    s = lax.dot_general(h, w2_ref[...], (((1,), (0,)), ((), ())),
                        preferred_element_type=jnp.float32) + b2_ref[...]
    rows = lax.broadcasted_iota(jnp.int32, (B, 1), 0) + i * B
    ex = jnp.where(rows < N, jnp.exp(s), 0.0)
    ex_ref[...] = ex

    idx_row = idx_ref[...].reshape(1, B)
    oht = (lax.broadcasted_iota(jnp.int32, (G, B), 0) == idx_row
           ).astype(jnp.float32)
    dpart = lax.dot_general(oht, ex, (((1,), (0,)), ((), ())),
                            preferred_element_type=jnp.float32)
    opart = lax.dot_general(oht, x * ex, (((1,), (0,)), ((), ())),
                            preferred_element_type=jnp.float32)

    @pl.when(i == 0)
    def _():
        dacc[...] = jnp.zeros_like(dacc)
        out_ref[...] = jnp.zeros_like(out_ref)

    dacc[...] += dpart
    out_ref[...] += opart

    @pl.when(i == NB - 1)
    def _():
        d = dacc[...]
        denom_ref[...] = d
        out_ref[...] = out_ref[...] / (d + 1e-16)


def _tc_call(xp, idx3, W1, b1r, W2, b2r):
    return pl.pallas_call(
        _tc_body,
        grid=(NB,),
        in_specs=[
            pl.BlockSpec((B, D), lambda i: (i, 0)),
            pl.BlockSpec((1, 1, B), lambda i: (i, 0, 0)),
            pl.BlockSpec((D, H), lambda i: (0, 0)),
            pl.BlockSpec((1, H), lambda i: (0, 0)),
            pl.BlockSpec((H, 1), lambda i: (0, 0)),
            pl.BlockSpec((1, 1), lambda i: (0, 0)),
        ],
        out_specs=[
            pl.BlockSpec((B, 1), lambda i: (i, 0)),
            pl.BlockSpec((G, D), lambda i: (0, 0)),
            pl.BlockSpec((G, 1), lambda i: (0, 0)),
        ],
        out_shape=[
            jax.ShapeDtypeStruct((NP, 1), jnp.float32),
            jax.ShapeDtypeStruct((G, D), jnp.float32),
            jax.ShapeDtypeStruct((G, 1), jnp.float32),
        ],
        scratch_shapes=[pltpu.VMEM((G, 1), jnp.float32)],
    )(xp, idx3, W1, b1r, W2, b2r)


@functools.cache
def _alpha_sc_build():
    @functools.partial(
        pl.kernel,
        out_type=jax.ShapeDtypeStruct((NP,), jnp.float32),
        mesh=plsc.VectorSubcoreMesh(core_axis_name="c", subcore_axis_name="s"),
        scratch_types=[
            pltpu.VMEM((CHUNK,), jnp.float32),
            pltpu.VMEM((CHUNK,), jnp.int32),
            pltpu.VMEM((CHUNK,), jnp.float32),
            pltpu.SemaphoreType.DMA,
        ],
    )
    def _alpha_sc(ex_hbm, idx_hbm, denom_hbm, alpha_hbm, ex_v, idx_v, dg_v,
                  sem):
        wid = lax.axis_index("s") * 2 + lax.axis_index("c")
        base = wid * CHUNK
        pltpu.sync_copy(idx_hbm.at[pl.ds(base, CHUNK)], idx_v)
        gat = pltpu.async_copy(denom_hbm.at[idx_v], dg_v, sem)
        pltpu.sync_copy(ex_hbm.at[pl.ds(base, CHUNK)], ex_v)
        gat.wait()

        def body(j, _):
            o = pl.multiple_of(j * LANES, LANES)
            ev = ex_v[pl.ds(o, LANES)]
            dv = dg_v[pl.ds(o, LANES)]
            ex_v[pl.ds(o, LANES)] = ev / (dv + 1e-16)
            return 0

        lax.fori_loop(0, CHUNK // LANES, body, 0)
        pltpu.sync_copy(ex_v, alpha_hbm.at[pl.ds(base, CHUNK)])

    return _alpha_sc


def kernel(x, batch_idx, dim_size, W1, b1, W2, b2):
    pad = NP - N
    xp = jnp.concatenate([x, jnp.zeros((pad, D), x.dtype)])
    idxp = jnp.concatenate(
        [batch_idx.astype(jnp.int32), jnp.zeros((pad,), jnp.int32)])
    idx3 = idxp.reshape(NB, 1, B)
    ex_p, out, denom = _tc_call(xp, idx3, W1, b1.reshape(1, H),
                                W2.reshape(H, 1), b2.reshape(1, 1))
    alpha_p = _alpha_sc_build()(ex_p.reshape(NP), idxp, denom.reshape(G))
    return out, alpha_p[:N].reshape(N, 1)
